# baseline (speedup 1.0000x reference)
import functools
import math

import jax
import jax.numpy as jnp
from jax import lax
from jax.experimental import pallas as pl
from jax.experimental.pallas import tpu as pltpu
from jax.experimental.pallas import tpu_sc as plsc

_VOCAB = 1000000
_EMB = 64
_SCALE = math.sqrt(_EMB)

_NC = 2
_NS = 16
_NW = _NC * _NS

_VB = 256
_NVB = _VOCAB // _VB
_NVB_EACH = _NVB // _NW
_NVB_LEFT = _NVB - _NVB_EACH * _NW
_VREM = _VOCAB - _NVB * _VB

_B = 1024 * 200 * 4
_BPW = _B // _NW
_CHUNK = 128
_NCHUNK = _BPW // _CHUNK
_NBUF = 3
_NGRP = _NCHUNK // _NBUF

_mesh = plsc.VectorSubcoreMesh(core_axis_name="c", subcore_axis_name="s")


@functools.partial(
    pl.kernel,
    mesh=_mesh,
    out_type=jax.ShapeDtypeStruct((_VOCAB // 2, 2 * _EMB), jnp.float32),
    scratch_types=[
        pltpu.VMEM((2, _EMB, _VB + 9), jnp.float32),
        pltpu.VMEM((2, _VB // 2, 2 * _EMB + 8), jnp.float32),
        pltpu.SemaphoreType.DMA((2,)),
        pltpu.SemaphoreType.DMA((2,)),
    ],
    compiler_params=pltpu.CompilerParams(
        use_tc_tiling_on_sc=True, needs_layout_passes=False),
)
def _tconv(tt_hbm, out_hbm, cin, cout, gsem, ssem):
    wid = lax.axis_index("s") * _NC + lax.axis_index("c")

    def stage_start(v0, width, b):
        v0 = pl.multiple_of(v0, 128)
        for te in range(8):
            pltpu.async_copy(
                tt_hbm.at[pl.ds(te * 8, 8), pl.ds(v0, width)],
                cin.at[b, pl.ds(te * 8, 8), pl.ds(0, width)], gsem.at[b])

    def stage_wait(width, b):
        for te in range(8):
            pltpu.make_async_copy(
                tt_hbm.at[pl.ds(0, 8), pl.ds(0, width)],
                cin.at[b, pl.ds(0, 8), pl.ds(0, width)], gsem.at[b]).wait()

    def transpose(width, b):
        ev = [lax.iota(jnp.int32, 16) + (t * 16) for t in range(_EMB // 16)]

        @plsc.parallel_loop(0, width // 2, unroll=8)
        def pair(k):
            for h in range(2):
                vv = jnp.zeros((16,), jnp.int32) + (2 * k + h)
                for t in range(_EMB // 16):
                    cout[b, k, pl.ds(h * _EMB + t * 16, 16)] = (
                        plsc.load_gather(cin.at[b], [ev[t], vv]))

    def store_start(v0, width, b):
        o0 = pl.multiple_of(v0 // 2, 64)
        pltpu.async_copy(
            cout.at[b, pl.ds(0, width // 2), pl.ds(0, 2 * _EMB)],
            out_hbm.at[pl.ds(o0, width // 2)], ssem.at[b])

    def store_wait(width, b):
        pltpu.make_async_copy(
            cout.at[b, pl.ds(0, width // 2), pl.ds(0, 2 * _EMB)],
            out_hbm.at[pl.ds(0, width // 2)], ssem.at[b]).wait()

    my_base = wid * _NVB_EACH
    extra_full = jnp.where(wid < _NVB_LEFT, 1, 0)
    has_rem = jnp.where(wid == _NW - 1, 1, 0)
    n_full = _NVB_EACH + extra_full

    def full_v0(g):
        v0_main = (my_base + g) * _VB
        v0_extra = (_NW * _NVB_EACH + wid) * _VB
        return jnp.where(g < _NVB_EACH, v0_main, v0_extra)

    stage_start(full_v0(0), _VB, 0)

    def grp(p, c):
        for b in range(2):
            g = 2 * p + b
            @pl.when(g + 1 < n_full)
            def _():
                stage_start(full_v0(g + 1), _VB, 1 - b)
            stage_wait(_VB, b)
            @pl.when(g >= 2)
            def _():
                store_wait(_VB, b)
            transpose(_VB, b)
            store_start(full_v0(g), _VB, b)
        return c

    lax.fori_loop(0, _NVB_EACH // 2, grp, 0)
    @pl.when(extra_full == 1)
    def _():
        g = _NVB_EACH
        stage_wait(_VB, 0)
        store_wait(_VB, 0)
        transpose(_VB, 0)
        store_start(full_v0(g), _VB, 0)
    for b in range(2):
        store_wait(_VB, b)

    @pl.when(has_rem == 1)
    def _():
        v0 = _NVB * _VB
        stage_start(v0, _VREM, 0)
        stage_wait(_VREM, 0)
        transpose(_VREM, 0)
        store_start(v0, _VREM, 0)
        store_wait(_VREM, 0)


@functools.partial(
    pl.kernel,
    mesh=_mesh,
    out_type=jax.ShapeDtypeStruct((_B, _EMB), jnp.float32),
    scratch_types=[
        pltpu.VMEM((_NCHUNK, _CHUNK), jnp.int32),
        pltpu.VMEM((_NBUF, _CHUNK, _EMB), jnp.float32),
        pltpu.VMEM((_NBUF, _CHUNK, _EMB), jnp.float32),
        pltpu.SemaphoreType.DMA((_NBUF,)),
        pltpu.SemaphoreType.DMA((_NBUF,)),
    ],
    compiler_params=pltpu.CompilerParams(use_tc_tiling_on_sc=False),
)
def _emb_lookup(idx_hbm, table_hbm, out_hbm, idx_v, gbuf, sbuf, gsem, ssem):
    wid = lax.axis_index("s") * _NC + lax.axis_index("c")
    base = wid * _BPW
    pltpu.sync_copy(idx_hbm.at[pl.ds(wid * _NCHUNK, _NCHUNK)], idx_v)

    def gather_start(j, b):
        pltpu.async_copy(table_hbm.at[idx_v.at[j]], gbuf.at[b], gsem.at[b])

    def gather_wait(b):
        pltpu.make_async_copy(
            table_hbm.at[idx_v.at[0]], gbuf.at[b], gsem.at[b]).wait()

    def scatter_start(j, b):
        pltpu.async_copy(
            sbuf.at[b], out_hbm.at[pl.ds(base + j * _CHUNK, _CHUNK)],
            ssem.at[b])

    def scatter_wait(b):
        pltpu.make_async_copy(
            sbuf.at[b], out_hbm.at[pl.ds(base, _CHUNK)], ssem.at[b]).wait()

    def scale(b):
        def row_body(i, c):
            for t in range(_EMB // 16):
                sl = pl.ds(t * 16, 16)
                sbuf[b, i, sl] = gbuf[b, i, sl] * _SCALE
            return c
        lax.fori_loop(0, _CHUNK, row_body, 0, unroll=8)

    for b in range(_NBUF):
        gather_start(b, b)

    def group(g, carry):
        for b in range(_NBUF):
            j = g * _NBUF + b
            gather_wait(b)
            @pl.when(g > 0)
            def _():
                scatter_wait(b)
            scale(b)
            @pl.when(j + _NBUF < _NCHUNK)
            def _():
                gather_start(j + _NBUF, b)
            scatter_start(j, b)
        return carry

    lax.fori_loop(0, _NGRP, group, 0)

    for j in range(_NGRP * _NBUF, _NCHUNK):
        b = j % _NBUF
        gather_wait(b)
        scatter_wait(b)
        scale(b)
        scatter_start(j, b)
    for b in range(_NBUF):
        scatter_wait(b)


def kernel(content, table):
    bs, l, sub = content.shape
    idx = content.astype(jnp.int32).reshape(_B // 128, 128)
    tconv = _tconv(table.T)
    tlin = tconv.reshape(_VOCAB, _EMB)
    out = _emb_lookup(idx, tlin)
    return out.reshape(bs, l, sub * _EMB)


# speedup vs baseline: 1.1106x; 1.1106x over previous
import functools
import math

import jax
import jax.numpy as jnp
from jax import lax
from jax.experimental import pallas as pl
from jax.experimental.pallas import tpu as pltpu
from jax.experimental.pallas import tpu_sc as plsc

_VOCAB = 1000000
_EMB = 64
_SCALE = math.sqrt(_EMB)

_NC = 2
_NS = 16
_NW = _NC * _NS

_RB = 320
_NRB = _VOCAB // _RB
_NRB_EACH = _NRB // _NW
_NRB_LEFT = _NRB - _NRB_EACH * _NW

_B = 1024 * 200 * 4
_BPW = _B // _NW
_CHUNK = 128
_NCHUNK = _BPW // _CHUNK
_NBUF = 3
_NGRP = _NCHUNK // _NBUF

_mesh = plsc.VectorSubcoreMesh(core_axis_name="c", subcore_axis_name="s")


@functools.partial(
    pl.kernel,
    mesh=_mesh,
    out_type=jax.ShapeDtypeStruct((_VOCAB // 2, 2 * _EMB), jnp.float32),
    scratch_types=[
        pltpu.VMEM((2, _RB, _EMB), jnp.float32),
        pltpu.VMEM((2, _RB // 2, 2 * _EMB), jnp.float32),
        pltpu.SemaphoreType.DMA((2,)),
        pltpu.SemaphoreType.DMA((2,)),
    ],
    compiler_params=pltpu.CompilerParams(
        use_tc_tiling_on_sc=True, needs_layout_passes=False),
)
def _tconv(t_hbm, out_hbm, cin, cout, gsem, ssem):
    wid = lax.axis_index("s") * _NC + lax.axis_index("c")

    def stage_start(r0, b):
        r0 = pl.multiple_of(r0, 8)
        pltpu.async_copy(t_hbm.at[pl.ds(r0, _RB)], cin.at[b], gsem.at[b])

    def stage_wait(b):
        pltpu.make_async_copy(
            t_hbm.at[pl.ds(0, _RB)], cin.at[b], gsem.at[b]).wait()

    def pack(b):
        @plsc.parallel_loop(0, _RB // 2, unroll=4)
        def pair(k):
            for h in range(2):
                for t in range(_EMB // 16):
                    cout[b, k, pl.ds(h * _EMB + t * 16, 16)] = (
                        cin[b, 2 * k + h, pl.ds(t * 16, 16)])

    def store_start(r0, b):
        o0 = pl.multiple_of(r0 // 2, 8)
        pltpu.async_copy(
            cout.at[b], out_hbm.at[pl.ds(o0, _RB // 2)], ssem.at[b])

    def store_wait(b):
        pltpu.make_async_copy(
            cout.at[b], out_hbm.at[pl.ds(0, _RB // 2)], ssem.at[b]).wait()

    my_base = wid * _NRB_EACH
    extra = jnp.where(wid < _NRB_LEFT, 1, 0)
    n_full = _NRB_EACH + extra

    def blk_r0(g):
        main = (my_base + g) * _RB
        left = (_NW * _NRB_EACH + wid) * _RB
        return jnp.where(g < _NRB_EACH, main, left)

    stage_start(blk_r0(0), 0)

    def grp(p, c):
        for b in range(2):
            g = 2 * p + b
            @pl.when(g + 1 < n_full)
            def _():
                stage_start(blk_r0(g + 1), 1 - b)
            stage_wait(b)
            @pl.when(g >= 2)
            def _():
                store_wait(b)
            pack(b)
            store_start(blk_r0(g), b)
        return c

    lax.fori_loop(0, (_NRB_EACH - 1) // 2, grp, 0)
    g96 = _NRB_EACH - 1
    @pl.when(g96 + 1 < n_full)
    def _():
        stage_start(blk_r0(g96 + 1), 1)
    stage_wait(0)
    store_wait(0)
    pack(0)
    store_start(blk_r0(g96), 0)
    @pl.when(extra == 1)
    def _():
        stage_wait(1)
        store_wait(1)
        pack(1)
        store_start(blk_r0(g96 + 1), 1)
    store_wait(0)
    store_wait(1)


@functools.partial(
    pl.kernel,
    mesh=_mesh,
    out_type=jax.ShapeDtypeStruct((_B, _EMB), jnp.float32),
    scratch_types=[
        pltpu.VMEM((_NCHUNK, _CHUNK), jnp.int32),
        pltpu.VMEM((_NBUF, _CHUNK, _EMB), jnp.float32),
        pltpu.VMEM((_NBUF, _CHUNK, _EMB), jnp.float32),
        pltpu.SemaphoreType.DMA((_NBUF,)),
        pltpu.SemaphoreType.DMA((_NBUF,)),
    ],
    compiler_params=pltpu.CompilerParams(use_tc_tiling_on_sc=False),
)
def _emb_lookup(idx_hbm, table_hbm, out_hbm, idx_v, gbuf, sbuf, gsem, ssem):
    wid = lax.axis_index("s") * _NC + lax.axis_index("c")
    base = wid * _BPW
    pltpu.sync_copy(idx_hbm.at[pl.ds(wid * _NCHUNK, _NCHUNK)], idx_v)

    def gather_start(j, b):
        pltpu.async_copy(table_hbm.at[idx_v.at[j]], gbuf.at[b], gsem.at[b])

    def gather_wait(b):
        pltpu.make_async_copy(
            table_hbm.at[idx_v.at[0]], gbuf.at[b], gsem.at[b]).wait()

    def scatter_start(j, b):
        pltpu.async_copy(
            sbuf.at[b], out_hbm.at[pl.ds(base + j * _CHUNK, _CHUNK)],
            ssem.at[b])

    def scatter_wait(b):
        pltpu.make_async_copy(
            sbuf.at[b], out_hbm.at[pl.ds(base, _CHUNK)], ssem.at[b]).wait()

    def scale(b):
        def row_body(i, c):
            for t in range(_EMB // 16):
                sl = pl.ds(t * 16, 16)
                sbuf[b, i, sl] = gbuf[b, i, sl] * _SCALE
            return c
        lax.fori_loop(0, _CHUNK, row_body, 0, unroll=8)

    for b in range(_NBUF):
        gather_start(b, b)

    def group(g, carry):
        for b in range(_NBUF):
            j = g * _NBUF + b
            gather_wait(b)
            @pl.when(g > 0)
            def _():
                scatter_wait(b)
            scale(b)
            @pl.when(j + _NBUF < _NCHUNK)
            def _():
                gather_start(j + _NBUF, b)
            scatter_start(j, b)
        return carry

    lax.fori_loop(0, _NGRP, group, 0)

    for j in range(_NGRP * _NBUF, _NCHUNK):
        b = j % _NBUF
        gather_wait(b)
        scatter_wait(b)
        scale(b)
        scatter_start(j, b)
    for b in range(_NBUF):
        scatter_wait(b)


def kernel(content, table):
    bs, l, sub = content.shape
    idx = content.astype(jnp.int32).reshape(_B // 128, 128)
    tconv = _tconv(table)
    tlin = tconv.reshape(_VOCAB, _EMB)
    out = _emb_lookup(idx, tlin)
    return out.reshape(bs, l, sub * _EMB)


# speedup vs baseline: 1.1804x; 1.0628x over previous
import functools
import math

import jax
import jax.numpy as jnp
from jax import lax
from jax.experimental import pallas as pl
from jax.experimental.pallas import tpu as pltpu
from jax.experimental.pallas import tpu_sc as plsc

_VOCAB = 1000000
_EMB = 64
_SCALE = math.sqrt(_EMB)

_NC = 2
_NS = 16
_NW = _NC * _NS

_RB = 320
_NRB = _VOCAB // _RB
_NRB_EACH = _NRB // _NW
_NRB_LEFT = _NRB - _NRB_EACH * _NW

_B = 1024 * 200 * 4
_BPW = _B // _NW
_CHUNK = 128
_NCHUNK = _BPW // _CHUNK
_NBUF = 4
_NGRP = _NCHUNK // _NBUF

_mesh = plsc.VectorSubcoreMesh(core_axis_name="c", subcore_axis_name="s")


@functools.partial(
    pl.kernel,
    mesh=_mesh,
    out_type=jax.ShapeDtypeStruct((_VOCAB // 2, 2 * _EMB), jnp.float32),
    scratch_types=[
        pltpu.VMEM((2, _RB, _EMB), jnp.float32),
        pltpu.VMEM((2, _RB // 2, 2 * _EMB), jnp.float32),
        pltpu.SemaphoreType.DMA((2,)),
        pltpu.SemaphoreType.DMA((2,)),
    ],
    compiler_params=pltpu.CompilerParams(
        use_tc_tiling_on_sc=True, needs_layout_passes=False),
)
def _tconv(t_hbm, out_hbm, cin, cout, gsem, ssem):
    wid = lax.axis_index("s") * _NC + lax.axis_index("c")

    def stage_start(r0, b):
        r0 = pl.multiple_of(r0, 8)
        pltpu.async_copy(t_hbm.at[pl.ds(r0, _RB)], cin.at[b], gsem.at[b])

    def stage_wait(b):
        pltpu.make_async_copy(
            t_hbm.at[pl.ds(0, _RB)], cin.at[b], gsem.at[b]).wait()

    def pack(b):
        @plsc.parallel_loop(0, _RB // 2, unroll=4)
        def pair(k):
            for h in range(2):
                for t in range(_EMB // 16):
                    cout[b, k, pl.ds(h * _EMB + t * 16, 16)] = (
                        cin[b, 2 * k + h, pl.ds(t * 16, 16)])

    def store_start(r0, b):
        o0 = pl.multiple_of(r0 // 2, 8)
        pltpu.async_copy(
            cout.at[b], out_hbm.at[pl.ds(o0, _RB // 2)], ssem.at[b])

    def store_wait(b):
        pltpu.make_async_copy(
            cout.at[b], out_hbm.at[pl.ds(0, _RB // 2)], ssem.at[b]).wait()

    my_base = wid * _NRB_EACH
    extra = jnp.where(wid < _NRB_LEFT, 1, 0)
    n_full = _NRB_EACH + extra

    def blk_r0(g):
        main = (my_base + g) * _RB
        left = (_NW * _NRB_EACH + wid) * _RB
        return jnp.where(g < _NRB_EACH, main, left)

    stage_start(blk_r0(0), 0)

    def grp(p, c):
        for b in range(2):
            g = 2 * p + b
            @pl.when(g + 1 < n_full)
            def _():
                stage_start(blk_r0(g + 1), 1 - b)
            stage_wait(b)
            @pl.when(g >= 2)
            def _():
                store_wait(b)
            pack(b)
            store_start(blk_r0(g), b)
        return c

    lax.fori_loop(0, (_NRB_EACH - 1) // 2, grp, 0)
    g96 = _NRB_EACH - 1
    @pl.when(g96 + 1 < n_full)
    def _():
        stage_start(blk_r0(g96 + 1), 1)
    stage_wait(0)
    store_wait(0)
    pack(0)
    store_start(blk_r0(g96), 0)
    @pl.when(extra == 1)
    def _():
        stage_wait(1)
        store_wait(1)
        pack(1)
        store_start(blk_r0(g96 + 1), 1)
    store_wait(0)
    store_wait(1)


@functools.partial(
    pl.kernel,
    mesh=_mesh,
    out_type=jax.ShapeDtypeStruct((_B, _EMB), jnp.float32),
    scratch_types=[
        pltpu.VMEM((_BPW,), jnp.int32),
        pltpu.VMEM((_NBUF, _CHUNK, _EMB), jnp.float32),
        pltpu.VMEM((_NBUF, _CHUNK, _EMB), jnp.float32),
        pltpu.SemaphoreType.DMA((_NBUF,)),
        pltpu.SemaphoreType.DMA((_NBUF,)),
    ],
    compiler_params=pltpu.CompilerParams(use_tc_tiling_on_sc=False),
)
def _emb_lookup(idx_hbm, table_hbm, out_hbm, idx_v, gbuf, sbuf, gsem, ssem):
    wid = lax.axis_index("s") * _NC + lax.axis_index("c")
    base = wid * _BPW
    pltpu.sync_copy(idx_hbm.at[pl.ds(base, _BPW)], idx_v)

    def gather_start(j, b):
        pltpu.async_copy(
            table_hbm.at[idx_v.at[pl.ds(j * _CHUNK, _CHUNK)]],
            gbuf.at[b], gsem.at[b])

    def gather_wait(b):
        pltpu.make_async_copy(
            table_hbm.at[idx_v.at[pl.ds(0, _CHUNK)]],
            gbuf.at[b], gsem.at[b]).wait()

    def scatter_start(j, b):
        pltpu.async_copy(
            sbuf.at[b], out_hbm.at[pl.ds(base + j * _CHUNK, _CHUNK)],
            ssem.at[b])

    def scatter_wait(b):
        pltpu.make_async_copy(
            sbuf.at[b], out_hbm.at[pl.ds(base, _CHUNK)], ssem.at[b]).wait()

    def scale(b):
        def row_body(i, c):
            for t in range(_EMB // 16):
                sl = pl.ds(t * 16, 16)
                sbuf[b, i, sl] = gbuf[b, i, sl] * _SCALE
            return c
        lax.fori_loop(0, _CHUNK, row_body, 0, unroll=8)

    for b in range(_NBUF):
        gather_start(b, b)

    def group(g, carry):
        for b in range(_NBUF):
            j = g * _NBUF + b
            gather_wait(b)
            @pl.when(g > 0)
            def _():
                scatter_wait(b)
            scale(b)
            @pl.when(j + _NBUF < _NCHUNK)
            def _():
                gather_start(j + _NBUF, b)
            scatter_start(j, b)
        return carry

    lax.fori_loop(0, _NGRP, group, 0)

    for j in range(_NGRP * _NBUF, _NCHUNK):
        b = j % _NBUF
        gather_wait(b)
        scatter_wait(b)
        scale(b)
        scatter_start(j, b)
    for b in range(_NBUF):
        scatter_wait(b)


def kernel(content, table):
    bs, l, sub = content.shape
    idx = content.astype(jnp.int32).reshape(_B)
    tconv = _tconv(table)
    tlin = tconv.reshape(_VOCAB, _EMB)
    out = _emb_lookup(idx, tlin)
    return out.reshape(bs, l, sub * _EMB)
